# baseline (speedup 1.0000x reference)
import functools

import jax
import jax.numpy as jnp
from jax import lax
from jax.experimental import pallas as pl
from jax.experimental.pallas import tpu as pltpu
from jax.experimental.pallas import tpu_sc as plsc

N = 10000
NP = 10240
E = 320000
NG = 8
NW = 32
EPW = E // NW
C = 80
NCHUNK = EPW // C
BK = 1024
G = NP // BK

_mesh = lambda: plsc.VectorSubcoreMesh(core_axis_name="c", subcore_axis_name="s")


@functools.partial(
    pl.kernel, mesh=_mesh(),
    out_type=jax.ShapeDtypeStruct((2, NP), jnp.float32),
    scratch_types=[
        pltpu.VMEM((NCHUNK, C), jnp.int32),
        pltpu.VMEM((C,), jnp.float32),
        pltpu.VMEM((640,), jnp.float32),
        pltpu.VMEM_SHARED((NP,), jnp.float32),
    ],
)
def _sc_degree(dst_hbm, out, idx_d, ones_v, zb, acc):
    c = lax.axis_index("c")
    s = lax.axis_index("s")
    blk = c * 16 + s
    pltpu.sync_copy(dst_hbm.at[blk], idx_d)
    for k in range(C // 16):
        ones_v[pl.ds(k * 16, 16)] = jnp.ones((16,), jnp.float32)
    for k in range(640 // 16):
        zb[pl.ds(k * 16, 16)] = jnp.zeros((16,), jnp.float32)
    pltpu.sync_copy(zb, acc.at[pl.ds(s * 640, 640)])
    plsc.subcore_barrier()

    def body(j, carry):
        pltpu.sync_copy(ones_v, acc.at[idx_d.at[j]], add=True)
        return carry

    lax.fori_loop(0, NCHUNK, body, 0)
    plsc.subcore_barrier()
    pltpu.sync_copy(acc.at[pl.ds(s * 640, 640)], out.at[c, pl.ds(s * 640, 640)])


def _make_sc_scatter(D, sch, spmem_table=False):
    nsup = NCHUNK // sch
    sds = jax.ShapeDtypeStruct((NP, D), jnp.float32)
    table_scratch = (
        [pltpu.VMEM_SHARED((NP, D), jnp.float32)] if spmem_table else [])

    @functools.partial(
        pl.kernel, mesh=_mesh(),
        out_type=(sds, sds),
        compiler_params=pltpu.CompilerParams(use_tc_tiling_on_sc=(D == 128)),
        scratch_types=[
            pltpu.VMEM((sch, C), jnp.int32),
            pltpu.VMEM((sch, C), jnp.int32),
            pltpu.VMEM((C, D), jnp.float32),
            pltpu.VMEM((C, D), jnp.float32),
            pltpu.VMEM_SHARED((NP, D), jnp.float32),
        ] + table_scratch + [
            pltpu.SemaphoreType.DMA,
            pltpu.SemaphoreType.DMA,
        ],
    )
    def k(y_hbm, src_hbm, dst_hbm, out0, out1, idx_s, idx_d, r0, r1, acc,
          *rest):
        if spmem_table:
            y_sh, sem0, sem1 = rest
        else:
            sem0, sem1 = rest
        c = lax.axis_index("c")
        s = lax.axis_index("s")
        blk = c * 16 + s

        def zrow(r, carry):
            for k2 in range(D // 16):
                r0[r, pl.ds(k2 * 16, 16)] = jnp.zeros((16,), jnp.float32)
            return carry

        lax.fori_loop(0, C, zrow, 0)
        for t in range(NP // 16 // C):
            pltpu.sync_copy(r0, acc.at[pl.ds(s * 640 + t * C, C)])
        if spmem_table:
            pltpu.sync_copy(y_hbm.at[pl.ds(s * 640, 640)],
                            y_sh.at[pl.ds(s * 640, 640)])
            tab = y_sh
        else:
            tab = y_hbm
        plsc.subcore_barrier()

        for u in range(nsup):
            pltpu.sync_copy(src_hbm.at[blk, u], idx_s)
            pltpu.sync_copy(dst_hbm.at[blk, u], idx_d)
            pltpu.async_copy(tab.at[idx_s.at[0]], r0, sem0)

            def body(m, carry):
                j0 = 2 * m
                pltpu.make_async_copy(tab.at[idx_s.at[j0]], r0, sem0).wait()
                pltpu.async_copy(tab.at[idx_s.at[j0 + 1]], r1, sem1)
                pltpu.sync_copy(r0, acc.at[idx_d.at[j0]], add=True)
                pltpu.make_async_copy(tab.at[idx_s.at[j0 + 1]], r1, sem1).wait()
                pltpu.async_copy(tab.at[idx_s.at[j0 + 2]], r0, sem0)
                pltpu.sync_copy(r1, acc.at[idx_d.at[j0 + 1]], add=True)
                return carry

            lax.fori_loop(0, (sch - 1) // 2, body, 0)
            pltpu.make_async_copy(tab.at[idx_s.at[sch - 1]], r0, sem0).wait()
            pltpu.sync_copy(r0, acc.at[idx_d.at[sch - 1]], add=True)

        plsc.subcore_barrier()

        @pl.when(c == 0)
        def _():
            pltpu.sync_copy(acc.at[pl.ds(s * 640, 640)], out0.at[pl.ds(s * 640, 640)])

        @pl.when(c == 1)
        def _():
            pltpu.sync_copy(acc.at[pl.ds(s * 640, 640)], out1.at[pl.ds(s * 640, 640)])

    return k


_sc_scatter128 = _make_sc_scatter(128, 25)
_sc_scatter64 = _make_sc_scatter(64, 125, spmem_table=True)


def _dinv_col(deg_ref):
    deg = deg_ref[0] + deg_ref[1] + 1.0
    return lax.rsqrt(deg)


def _xw1_body(x_ref, w_ref, xw_ref):
    xw_ref[...] = jnp.dot(x_ref[...], w_ref[...],
                          preferred_element_type=jnp.float32)


def _scale_body(xw_ref, deg_ref, y_ref):
    y_ref[...] = xw_ref[...] * _dinv_col(deg_ref)


def _y2_body(a0_ref, a1_ref, y1_ref, deg_ref, b1_ref, w2_ref, y2_ref):
    dinv = _dinv_col(deg_ref)
    h = dinv * (a0_ref[...] + a1_ref[...] + y1_ref[...]) + b1_ref[...]
    h = jnp.maximum(h, 0.0)
    y2_ref[...] = jnp.dot(h, w2_ref[...], preferred_element_type=jnp.float32) * dinv


def _final_body(a0_ref, a1_ref, y2_ref, deg_ref, b2_ref, batch_ref, lat_ref,
                sum_ref, cnt_ref):
    i = pl.program_id(0)
    dinv = _dinv_col(deg_ref)
    out2 = dinv * (a0_ref[...] + a1_ref[...] + y2_ref[...]) + b2_ref[...]
    bb = batch_ref[...]
    gid = lax.broadcasted_iota(jnp.int32, (1, NG), 1).astype(jnp.float32)
    oht = (bb == gid).astype(jnp.float32)
    dn = (((0,), (0,)), ((), ()))
    contrib = lax.dot_general(oht, out2, dn,
                              preferred_element_type=jnp.float32)
    cnt = lax.dot_general(oht, jnp.ones((BK, 1), jnp.float32), dn,
                          preferred_element_type=jnp.float32)

    @pl.when(i == 0)
    def _():
        sum_ref[...] = jnp.zeros_like(sum_ref)
        cnt_ref[...] = jnp.zeros_like(cnt_ref)

    sum_ref[...] += contrib
    cnt_ref[...] += jnp.broadcast_to(cnt, (NG, 64))

    @pl.when(i == G - 1)
    def _():
        lat_ref[...] = sum_ref[...] / jnp.maximum(cnt_ref[...], 1.0)


_row_spec = lambda d: pl.BlockSpec((BK, d), lambda i: (i, 0))
_deg_spec = pl.BlockSpec((2, BK, 1), lambda i: (0, i, 0))


def _full(shape):
    return pl.BlockSpec(shape, lambda *_: tuple(0 for _ in shape))


_xw1_call = pl.pallas_call(
    _xw1_body, grid=(G,),
    in_specs=[_row_spec(128), _full((128, 128))],
    out_specs=_row_spec(128),
    out_shape=jax.ShapeDtypeStruct((NP, 128), jnp.float32),
)

_scale_call = pl.pallas_call(
    _scale_body, grid=(G,),
    in_specs=[_row_spec(128), _deg_spec],
    out_specs=_row_spec(128),
    out_shape=jax.ShapeDtypeStruct((NP, 128), jnp.float32),
)

_y2_call = pl.pallas_call(
    _y2_body, grid=(G,),
    in_specs=[_row_spec(128), _row_spec(128), _row_spec(128), _deg_spec,
              _full((1, 128)), _full((128, 64))],
    out_specs=_row_spec(64),
    out_shape=jax.ShapeDtypeStruct((NP, 64), jnp.float32),
)

_final_call = pl.pallas_call(
    _final_body, grid=(G,),
    in_specs=[_row_spec(64), _row_spec(64), _row_spec(64), _deg_spec,
              _full((1, 64)), pl.BlockSpec((BK, 1), lambda i: (i, 0))],
    out_specs=_full((NG, 64)),
    out_shape=jax.ShapeDtypeStruct((NG, 64), jnp.float32),
    scratch_shapes=[pltpu.VMEM((NG, 64), jnp.float32),
                    pltpu.VMEM((NG, 64), jnp.float32)],
)


def kernel(x, edge_index, batch, W1, b1, W2, b2):
    x = x.astype(jnp.float32)
    ei = edge_index.astype(jnp.int32)
    src_a = ei[0].reshape(NW, NCHUNK // 25, 25, C)
    dst_a = ei[1].reshape(NW, NCHUNK // 25, 25, C)
    src_b = ei[0].reshape(NW, 1, NCHUNK, C)
    dst_b = ei[1].reshape(NW, 1, NCHUNK, C)
    dst_flat = ei[1].reshape(NW, NCHUNK, C)
    x_p = jnp.pad(x, ((0, NP - N), (0, 0)))
    batchf = jnp.pad(batch.astype(jnp.float32), (0, NP - N),
                     constant_values=float(NG)).reshape(NP, 1)

    degp = _sc_degree(dst_flat).reshape(2, NP, 1)
    y1 = _scale_call(_xw1_call(x_p, W1), degp)
    a10, a11 = _sc_scatter128(y1, src_a, dst_a)
    y2 = _y2_call(a10, a11, y1, degp, b1.reshape(1, 128), W2)
    a20, a21 = _sc_scatter64(y2, src_b, dst_b)
    return _final_call(a20, a21, y2, degp, b2.reshape(1, 64), batchf)


# speedup vs baseline: 1.0296x; 1.0296x over previous
import functools

import jax
import jax.numpy as jnp
from jax import lax
from jax.experimental import pallas as pl
from jax.experimental.pallas import tpu as pltpu
from jax.experimental.pallas import tpu_sc as plsc

N = 10000
NP = 10240
E = 320000
NG = 8
NW = 32
EPW = E // NW
C = 80
NCHUNK = EPW // C
BK = 1024
G = NP // BK

_mesh = lambda: plsc.VectorSubcoreMesh(core_axis_name="c", subcore_axis_name="s")


@functools.partial(
    pl.kernel, mesh=_mesh(),
    out_type=jax.ShapeDtypeStruct((2, NP), jnp.float32),
    scratch_types=[
        pltpu.VMEM((NCHUNK, C), jnp.int32),
        pltpu.VMEM((C,), jnp.float32),
        pltpu.VMEM((640,), jnp.float32),
        pltpu.VMEM_SHARED((NP,), jnp.float32),
        pltpu.SemaphoreType.DMA,
    ],
)
def _sc_degree(dst_hbm, out, idx_d, ones_v, zb, acc, dsem):
    c = lax.axis_index("c")
    s = lax.axis_index("s")
    blk = c * 16 + s
    pltpu.sync_copy(dst_hbm.at[blk], idx_d)
    for k in range(C // 16):
        ones_v[pl.ds(k * 16, 16)] = jnp.ones((16,), jnp.float32)
    for k in range(640 // 16):
        zb[pl.ds(k * 16, 16)] = jnp.zeros((16,), jnp.float32)
    pltpu.sync_copy(zb, acc.at[pl.ds(s * 640, 640)])
    plsc.subcore_barrier()

    W = 8

    def body(j, carry):
        pltpu.async_copy(ones_v, acc.at[idx_d.at[j]], dsem, add=True)

        @pl.when(j >= W)
        def _():
            pltpu.make_async_copy(ones_v, acc.at[idx_d.at[0]], dsem).wait()

        return carry

    lax.fori_loop(0, NCHUNK, body, 0)
    for _ in range(W):
        pltpu.make_async_copy(ones_v, acc.at[idx_d.at[0]], dsem).wait()
    plsc.subcore_barrier()
    pltpu.sync_copy(acc.at[pl.ds(s * 640, 640)], out.at[c, pl.ds(s * 640, 640)])


def _make_sc_scatter(D, sch, spmem_table=False):
    nsup = NCHUNK // sch
    sds = jax.ShapeDtypeStruct((NP, D), jnp.float32)
    table_scratch = (
        [pltpu.VMEM_SHARED((NP, D), jnp.float32)] if spmem_table else [])

    @functools.partial(
        pl.kernel, mesh=_mesh(),
        out_type=(sds, sds),
        compiler_params=pltpu.CompilerParams(use_tc_tiling_on_sc=(D == 128)),
        scratch_types=[
            pltpu.VMEM((sch, C), jnp.int32),
            pltpu.VMEM((sch, C), jnp.int32),
            pltpu.VMEM((C, D), jnp.float32),
            pltpu.VMEM((C, D), jnp.float32),
            pltpu.VMEM_SHARED((NP, D), jnp.float32),
        ] + table_scratch + [
            pltpu.SemaphoreType.DMA,
            pltpu.SemaphoreType.DMA,
        ],
    )
    def k(y_hbm, src_hbm, dst_hbm, out0, out1, idx_s, idx_d, r0, r1, acc,
          *rest):
        if spmem_table:
            y_sh, sem0, sem1 = rest
        else:
            sem0, sem1 = rest
        c = lax.axis_index("c")
        s = lax.axis_index("s")
        blk = c * 16 + s

        def zrow(r, carry):
            for k2 in range(D // 16):
                r0[r, pl.ds(k2 * 16, 16)] = jnp.zeros((16,), jnp.float32)
            return carry

        lax.fori_loop(0, C, zrow, 0)
        for t in range(NP // 16 // C):
            pltpu.sync_copy(r0, acc.at[pl.ds(s * 640 + t * C, C)])
        if spmem_table:
            pltpu.sync_copy(y_hbm.at[pl.ds(s * 640, 640)],
                            y_sh.at[pl.ds(s * 640, 640)])
            tab = y_sh
        else:
            tab = y_hbm
        plsc.subcore_barrier()

        for u in range(nsup):
            pltpu.sync_copy(src_hbm.at[blk, u], idx_s)
            pltpu.sync_copy(dst_hbm.at[blk, u], idx_d)
            pltpu.async_copy(tab.at[idx_s.at[0]], r0, sem0)

            def body(m, carry):
                j0 = 2 * m
                pltpu.make_async_copy(tab.at[idx_s.at[j0]], r0, sem0).wait()
                pltpu.async_copy(tab.at[idx_s.at[j0 + 1]], r1, sem1)
                pltpu.sync_copy(r0, acc.at[idx_d.at[j0]], add=True)
                pltpu.make_async_copy(tab.at[idx_s.at[j0 + 1]], r1, sem1).wait()
                pltpu.async_copy(tab.at[idx_s.at[j0 + 2]], r0, sem0)
                pltpu.sync_copy(r1, acc.at[idx_d.at[j0 + 1]], add=True)
                return carry

            lax.fori_loop(0, (sch - 1) // 2, body, 0)
            pltpu.make_async_copy(tab.at[idx_s.at[sch - 1]], r0, sem0).wait()
            pltpu.sync_copy(r0, acc.at[idx_d.at[sch - 1]], add=True)

        plsc.subcore_barrier()

        @pl.when(c == 0)
        def _():
            pltpu.sync_copy(acc.at[pl.ds(s * 640, 640)], out0.at[pl.ds(s * 640, 640)])

        @pl.when(c == 1)
        def _():
            pltpu.sync_copy(acc.at[pl.ds(s * 640, 640)], out1.at[pl.ds(s * 640, 640)])

    return k


_sc_scatter128 = _make_sc_scatter(128, 25)
_sc_scatter64 = _make_sc_scatter(64, 125, spmem_table=True)


def _dinv_col(deg_ref):
    deg = deg_ref[0] + deg_ref[1] + 1.0
    return lax.rsqrt(deg)


def _y1_body(x_ref, w_ref, deg_ref, y_ref):
    dinv = _dinv_col(deg_ref)
    xw = jnp.dot(x_ref[...], w_ref[...], preferred_element_type=jnp.float32)
    y_ref[...] = xw * dinv


def _y2_body(a0_ref, a1_ref, y1_ref, deg_ref, b1_ref, w2_ref, y2_ref):
    dinv = _dinv_col(deg_ref)
    h = dinv * (a0_ref[...] + a1_ref[...] + y1_ref[...]) + b1_ref[...]
    h = jnp.maximum(h, 0.0)
    y2_ref[...] = jnp.dot(h, w2_ref[...], preferred_element_type=jnp.float32) * dinv


def _final_body(a0_ref, a1_ref, y2_ref, deg_ref, b2_ref, batch_ref, lat_ref,
                sum_ref, cnt_ref):
    i = pl.program_id(0)
    dinv = _dinv_col(deg_ref)
    out2 = dinv * (a0_ref[...] + a1_ref[...] + y2_ref[...]) + b2_ref[...]
    bb = batch_ref[...]
    gid = lax.broadcasted_iota(jnp.int32, (1, NG), 1).astype(jnp.float32)
    oht = (bb == gid).astype(jnp.float32)
    dn = (((0,), (0,)), ((), ()))
    contrib = lax.dot_general(oht, out2, dn,
                              preferred_element_type=jnp.float32)
    cnt = lax.dot_general(oht, jnp.ones((BK, 1), jnp.float32), dn,
                          preferred_element_type=jnp.float32)

    @pl.when(i == 0)
    def _():
        sum_ref[...] = jnp.zeros_like(sum_ref)
        cnt_ref[...] = jnp.zeros_like(cnt_ref)

    sum_ref[...] += contrib
    cnt_ref[...] += jnp.broadcast_to(cnt, (NG, 64))

    @pl.when(i == G - 1)
    def _():
        lat_ref[...] = sum_ref[...] / jnp.maximum(cnt_ref[...], 1.0)


_row_spec = lambda d: pl.BlockSpec((BK, d), lambda i: (i, 0))
_deg_spec = pl.BlockSpec((2, BK, 1), lambda i: (0, i, 0))


def _full(shape):
    return pl.BlockSpec(shape, lambda *_: tuple(0 for _ in shape))


_y1_call = pl.pallas_call(
    _y1_body, grid=(G,),
    in_specs=[_row_spec(128), _full((128, 128)), _deg_spec],
    out_specs=_row_spec(128),
    out_shape=jax.ShapeDtypeStruct((NP, 128), jnp.float32),
)

_y2_call = pl.pallas_call(
    _y2_body, grid=(G,),
    in_specs=[_row_spec(128), _row_spec(128), _row_spec(128), _deg_spec,
              _full((1, 128)), _full((128, 64))],
    out_specs=_row_spec(64),
    out_shape=jax.ShapeDtypeStruct((NP, 64), jnp.float32),
)

_final_call = pl.pallas_call(
    _final_body, grid=(G,),
    in_specs=[_row_spec(64), _row_spec(64), _row_spec(64), _deg_spec,
              _full((1, 64)), pl.BlockSpec((BK, 1), lambda i: (i, 0))],
    out_specs=_full((NG, 64)),
    out_shape=jax.ShapeDtypeStruct((NG, 64), jnp.float32),
    scratch_shapes=[pltpu.VMEM((NG, 64), jnp.float32),
                    pltpu.VMEM((NG, 64), jnp.float32)],
)


def kernel(x, edge_index, batch, W1, b1, W2, b2):
    x = x.astype(jnp.float32)
    ei = edge_index.astype(jnp.int32)
    src_a = ei[0].reshape(NW, NCHUNK // 25, 25, C)
    dst_a = ei[1].reshape(NW, NCHUNK // 25, 25, C)
    src_b = ei[0].reshape(NW, 1, NCHUNK, C)
    dst_b = ei[1].reshape(NW, 1, NCHUNK, C)
    dst_flat = ei[1].reshape(NW, NCHUNK, C)
    x_p = jnp.pad(x, ((0, NP - N), (0, 0)))
    batchf = jnp.pad(batch.astype(jnp.float32), (0, NP - N),
                     constant_values=float(NG)).reshape(NP, 1)

    degp = _sc_degree(dst_flat).reshape(2, NP, 1)
    y1 = _y1_call(x_p, W1, degp)
    a10, a11 = _sc_scatter128(y1, src_a, dst_a)
    y2 = _y2_call(a10, a11, y1, degp, b1.reshape(1, 128), W2)
    a20, a21 = _sc_scatter64(y2, src_b, dst_b)
    return _final_call(a20, a21, y2, degp, b2.reshape(1, 64), batchf)


# speedup vs baseline: 1.0337x; 1.0040x over previous
import functools

import jax
import jax.numpy as jnp
from jax import lax
from jax.experimental import pallas as pl
from jax.experimental.pallas import tpu as pltpu
from jax.experimental.pallas import tpu_sc as plsc

N = 10000
NP = 10240
E = 320000
NG = 8
NW = 32
EPW = E // NW
C = 80
NCHUNK = EPW // C
BK = 1024
G = NP // BK

_mesh = lambda: plsc.VectorSubcoreMesh(core_axis_name="c", subcore_axis_name="s")


@functools.partial(
    pl.kernel, mesh=_mesh(),
    out_type=jax.ShapeDtypeStruct((2, NP), jnp.float32),
    scratch_types=[
        pltpu.VMEM((NCHUNK, C), jnp.int32),
        pltpu.VMEM((C,), jnp.float32),
        pltpu.VMEM((640,), jnp.float32),
        pltpu.VMEM_SHARED((NP,), jnp.float32),
        pltpu.SemaphoreType.DMA,
    ],
)
def _sc_degree(dst_hbm, out, idx_d, ones_v, zb, acc, dsem):
    c = lax.axis_index("c")
    s = lax.axis_index("s")
    blk = c * 16 + s
    pltpu.sync_copy(dst_hbm.at[blk], idx_d)
    for k in range(C // 16):
        ones_v[pl.ds(k * 16, 16)] = jnp.ones((16,), jnp.float32)
    for k in range(640 // 16):
        zb[pl.ds(k * 16, 16)] = jnp.zeros((16,), jnp.float32)
    pltpu.sync_copy(zb, acc.at[pl.ds(s * 640, 640)])
    plsc.subcore_barrier()

    W = 8

    def body(j, carry):
        pltpu.async_copy(ones_v, acc.at[idx_d.at[j]], dsem, add=True)

        @pl.when(j >= W)
        def _():
            pltpu.make_async_copy(ones_v, acc.at[idx_d.at[0]], dsem).wait()

        return carry

    lax.fori_loop(0, NCHUNK, body, 0)
    for _ in range(W):
        pltpu.make_async_copy(ones_v, acc.at[idx_d.at[0]], dsem).wait()
    plsc.subcore_barrier()
    pltpu.sync_copy(acc.at[pl.ds(s * 640, 640)], out.at[c, pl.ds(s * 640, 640)])


def _make_sc_scatter(D, sch, spmem_table=False):
    nsup = NCHUNK // sch
    sds = jax.ShapeDtypeStruct((NP, D), jnp.float32)
    table_scratch = (
        [pltpu.VMEM_SHARED((NP, D), jnp.float32)] if spmem_table else [])

    @functools.partial(
        pl.kernel, mesh=_mesh(),
        out_type=(sds, sds),
        compiler_params=pltpu.CompilerParams(use_tc_tiling_on_sc=(D == 128)),
        scratch_types=[
            pltpu.VMEM((sch, C), jnp.int32),
            pltpu.VMEM((sch, C), jnp.int32),
            pltpu.VMEM((C, D), jnp.float32),
            pltpu.VMEM((C, D), jnp.float32),
            pltpu.VMEM_SHARED((NP, D), jnp.float32),
        ] + table_scratch + [
            pltpu.SemaphoreType.DMA,
            pltpu.SemaphoreType.DMA,
        ],
    )
    def k(y_hbm, src_hbm, dst_hbm, out0, out1, idx_s, idx_d, r0, r1, acc,
          *rest):
        if spmem_table:
            y_sh, sem0, sem1 = rest
        else:
            sem0, sem1 = rest
        c = lax.axis_index("c")
        s = lax.axis_index("s")
        blk = c * 16 + s

        def zrow(r, carry):
            for k2 in range(D // 16):
                r0[r, pl.ds(k2 * 16, 16)] = jnp.zeros((16,), jnp.float32)
            return carry

        lax.fori_loop(0, C, zrow, 0)
        if spmem_table:
            pltpu.async_copy(y_hbm.at[pl.ds(s * 640, 640)],
                             y_sh.at[pl.ds(s * 640, 640)], sem1)
            tab = y_sh
        else:
            tab = y_hbm
        for t in range(NP // 16 // C):
            pltpu.async_copy(r0, acc.at[pl.ds(s * 640 + t * C, C)], sem0)
        for t in range(NP // 16 // C):
            pltpu.make_async_copy(r0, acc.at[pl.ds(s * 640, C)], sem0).wait()
        if spmem_table:
            pltpu.make_async_copy(y_hbm.at[pl.ds(s * 640, 640)],
                                  y_sh.at[pl.ds(s * 640, 640)], sem1).wait()
        plsc.subcore_barrier()

        for u in range(nsup):
            pltpu.sync_copy(src_hbm.at[blk, u], idx_s)
            pltpu.sync_copy(dst_hbm.at[blk, u], idx_d)
            pltpu.async_copy(tab.at[idx_s.at[0]], r0, sem0)

            def body(m, carry):
                j0 = 2 * m
                pltpu.make_async_copy(tab.at[idx_s.at[j0]], r0, sem0).wait()
                pltpu.async_copy(tab.at[idx_s.at[j0 + 1]], r1, sem1)
                pltpu.sync_copy(r0, acc.at[idx_d.at[j0]], add=True)
                pltpu.make_async_copy(tab.at[idx_s.at[j0 + 1]], r1, sem1).wait()
                pltpu.async_copy(tab.at[idx_s.at[j0 + 2]], r0, sem0)
                pltpu.sync_copy(r1, acc.at[idx_d.at[j0 + 1]], add=True)
                return carry

            lax.fori_loop(0, (sch - 1) // 2, body, 0)
            pltpu.make_async_copy(tab.at[idx_s.at[sch - 1]], r0, sem0).wait()
            pltpu.sync_copy(r0, acc.at[idx_d.at[sch - 1]], add=True)

        plsc.subcore_barrier()

        @pl.when(c == 0)
        def _():
            pltpu.sync_copy(acc.at[pl.ds(s * 640, 640)], out0.at[pl.ds(s * 640, 640)])

        @pl.when(c == 1)
        def _():
            pltpu.sync_copy(acc.at[pl.ds(s * 640, 640)], out1.at[pl.ds(s * 640, 640)])

    return k


_sc_scatter128 = _make_sc_scatter(128, 25)
_sc_scatter64 = _make_sc_scatter(64, 125, spmem_table=True)


def _dinv_col(deg_ref):
    deg = deg_ref[0] + deg_ref[1] + 1.0
    return lax.rsqrt(deg)


def _y1_body(x_ref, w_ref, deg_ref, y_ref):
    dinv = _dinv_col(deg_ref)
    xw = jnp.dot(x_ref[...], w_ref[...], preferred_element_type=jnp.float32)
    y_ref[...] = xw * dinv


def _y2_body(a0_ref, a1_ref, y1_ref, deg_ref, b1_ref, w2_ref, y2_ref):
    dinv = _dinv_col(deg_ref)
    h = dinv * (a0_ref[...] + a1_ref[...] + y1_ref[...]) + b1_ref[...]
    h = jnp.maximum(h, 0.0)
    y2_ref[...] = jnp.dot(h, w2_ref[...], preferred_element_type=jnp.float32) * dinv


def _final_body(a0_ref, a1_ref, y2_ref, deg_ref, b2_ref, batch_ref, lat_ref,
                sum_ref, cnt_ref):
    i = pl.program_id(0)
    dinv = _dinv_col(deg_ref)
    out2 = dinv * (a0_ref[...] + a1_ref[...] + y2_ref[...]) + b2_ref[...]
    bb = batch_ref[...]
    gid = lax.broadcasted_iota(jnp.int32, (1, NG), 1).astype(jnp.float32)
    oht = (bb == gid).astype(jnp.float32)
    dn = (((0,), (0,)), ((), ()))
    contrib = lax.dot_general(oht, out2, dn,
                              preferred_element_type=jnp.float32)
    cnt = lax.dot_general(oht, jnp.ones((BK, 1), jnp.float32), dn,
                          preferred_element_type=jnp.float32)

    @pl.when(i == 0)
    def _():
        sum_ref[...] = jnp.zeros_like(sum_ref)
        cnt_ref[...] = jnp.zeros_like(cnt_ref)

    sum_ref[...] += contrib
    cnt_ref[...] += jnp.broadcast_to(cnt, (NG, 64))

    @pl.when(i == G - 1)
    def _():
        lat_ref[...] = sum_ref[...] / jnp.maximum(cnt_ref[...], 1.0)


_row_spec = lambda d: pl.BlockSpec((BK, d), lambda i: (i, 0))
_deg_spec = pl.BlockSpec((2, BK, 1), lambda i: (0, i, 0))


def _full(shape):
    return pl.BlockSpec(shape, lambda *_: tuple(0 for _ in shape))


_y1_call = pl.pallas_call(
    _y1_body, grid=(G,),
    in_specs=[_row_spec(128), _full((128, 128)), _deg_spec],
    out_specs=_row_spec(128),
    out_shape=jax.ShapeDtypeStruct((NP, 128), jnp.float32),
)

_y2_call = pl.pallas_call(
    _y2_body, grid=(G,),
    in_specs=[_row_spec(128), _row_spec(128), _row_spec(128), _deg_spec,
              _full((1, 128)), _full((128, 64))],
    out_specs=_row_spec(64),
    out_shape=jax.ShapeDtypeStruct((NP, 64), jnp.float32),
)

_final_call = pl.pallas_call(
    _final_body, grid=(G,),
    in_specs=[_row_spec(64), _row_spec(64), _row_spec(64), _deg_spec,
              _full((1, 64)), pl.BlockSpec((BK, 1), lambda i: (i, 0))],
    out_specs=_full((NG, 64)),
    out_shape=jax.ShapeDtypeStruct((NG, 64), jnp.float32),
    scratch_shapes=[pltpu.VMEM((NG, 64), jnp.float32),
                    pltpu.VMEM((NG, 64), jnp.float32)],
)


def kernel(x, edge_index, batch, W1, b1, W2, b2):
    x = x.astype(jnp.float32)
    ei = edge_index.astype(jnp.int32)
    src_a = ei[0].reshape(NW, NCHUNK // 25, 25, C)
    dst_a = ei[1].reshape(NW, NCHUNK // 25, 25, C)
    src_b = ei[0].reshape(NW, 1, NCHUNK, C)
    dst_b = ei[1].reshape(NW, 1, NCHUNK, C)
    dst_flat = ei[1].reshape(NW, NCHUNK, C)
    x_p = jnp.pad(x, ((0, NP - N), (0, 0)))
    batchf = jnp.pad(batch.astype(jnp.float32), (0, NP - N),
                     constant_values=float(NG)).reshape(NP, 1)

    degp = _sc_degree(dst_flat).reshape(2, NP, 1)
    y1 = _y1_call(x_p, W1, degp)
    a10, a11 = _sc_scatter128(y1, src_a, dst_a)
    y2 = _y2_call(a10, a11, y1, degp, b1.reshape(1, 128), W2)
    a20, a21 = _sc_scatter64(y2, src_b, dst_b)
    return _final_call(a20, a21, y2, degp, b2.reshape(1, 64), batchf)


# speedup vs baseline: 1.0541x; 1.0197x over previous
import functools

import jax
import jax.numpy as jnp
from jax import lax
from jax.experimental import pallas as pl
from jax.experimental.pallas import tpu as pltpu
from jax.experimental.pallas import tpu_sc as plsc

N = 10000
NP = 10240
E = 320000
NG = 8
NW = 32
EPW = E // NW
C = 80
NCHUNK = EPW // C
BK = 1024
G = NP // BK

_mesh = lambda: plsc.VectorSubcoreMesh(core_axis_name="c", subcore_axis_name="s")


@functools.partial(
    pl.kernel, mesh=_mesh(),
    out_type=jax.ShapeDtypeStruct((2, NP), jnp.float32),
    scratch_types=[
        pltpu.VMEM((NCHUNK, C), jnp.int32),
        pltpu.VMEM((C,), jnp.float32),
        pltpu.VMEM((640,), jnp.float32),
        pltpu.VMEM_SHARED((NP,), jnp.float32),
        pltpu.SemaphoreType.DMA,
    ],
)
def _sc_degree(dst_hbm, out, idx_d, ones_v, zb, acc, dsem):
    c = lax.axis_index("c")
    s = lax.axis_index("s")
    blk = c * 16 + s
    pltpu.sync_copy(dst_hbm.at[blk], idx_d)
    for k in range(C // 16):
        ones_v[pl.ds(k * 16, 16)] = jnp.ones((16,), jnp.float32)
    for k in range(640 // 16):
        zb[pl.ds(k * 16, 16)] = jnp.zeros((16,), jnp.float32)
    pltpu.sync_copy(zb, acc.at[pl.ds(s * 640, 640)])
    plsc.subcore_barrier()

    W = 8

    def body(j, carry):
        pltpu.async_copy(ones_v, acc.at[idx_d.at[j]], dsem, add=True)

        @pl.when(j >= W)
        def _():
            pltpu.make_async_copy(ones_v, acc.at[idx_d.at[0]], dsem).wait()

        return carry

    lax.fori_loop(0, NCHUNK, body, 0)
    for _ in range(W):
        pltpu.make_async_copy(ones_v, acc.at[idx_d.at[0]], dsem).wait()
    plsc.subcore_barrier()
    pltpu.sync_copy(acc.at[pl.ds(s * 640, 640)], out.at[c, pl.ds(s * 640, 640)])


def _make_sc_scatter(D, sch, spmem_table=False):
    nsup = NCHUNK // sch
    sds = jax.ShapeDtypeStruct((NP, D), jnp.float32)
    table_scratch = (
        [pltpu.VMEM_SHARED((NP, D), jnp.float32)] if spmem_table else [])
    idx_scratch = [pltpu.VMEM((sch, C), jnp.int32)] * (2 if nsup == 1 else 4)

    @functools.partial(
        pl.kernel, mesh=_mesh(),
        out_type=(sds, sds),
        compiler_params=pltpu.CompilerParams(use_tc_tiling_on_sc=(D == 128)),
        scratch_types=idx_scratch + [
            pltpu.VMEM((C, D), jnp.float32),
            pltpu.VMEM((C, D), jnp.float32),
            pltpu.VMEM_SHARED((NP, D), jnp.float32),
        ] + table_scratch + [
            pltpu.SemaphoreType.DMA,
            pltpu.SemaphoreType.DMA,
            pltpu.SemaphoreType.DMA,
        ],
    )
    def k(y_hbm, src_hbm, dst_hbm, out0, out1, *refs):
        if nsup == 1:
            idx_bufs = [(refs[0], refs[1])] * 2
            rest = refs[2:]
        else:
            idx_bufs = [(refs[0], refs[1]), (refs[2], refs[3])]
            rest = refs[4:]
        if spmem_table:
            r0, r1, acc, y_sh, sem0, sem1, isem = rest
        else:
            r0, r1, acc, sem0, sem1, isem = rest
        c = lax.axis_index("c")
        s = lax.axis_index("s")
        blk = c * 16 + s

        def zrow(r, carry):
            for k2 in range(D // 16):
                r0[r, pl.ds(k2 * 16, 16)] = jnp.zeros((16,), jnp.float32)
            return carry

        lax.fori_loop(0, C, zrow, 0)
        if spmem_table:
            pltpu.async_copy(y_hbm.at[pl.ds(s * 640, 640)],
                             y_sh.at[pl.ds(s * 640, 640)], sem1)
            tab = y_sh
        else:
            tab = y_hbm
        for t in range(NP // 16 // C):
            pltpu.async_copy(r0, acc.at[pl.ds(s * 640 + t * C, C)], sem0)
        for t in range(NP // 16 // C):
            pltpu.make_async_copy(r0, acc.at[pl.ds(s * 640, C)], sem0).wait()
        if spmem_table:
            pltpu.make_async_copy(y_hbm.at[pl.ds(s * 640, 640)],
                                  y_sh.at[pl.ds(s * 640, 640)], sem1).wait()
        plsc.subcore_barrier()

        for u in range(nsup):
            cs, cd = idx_bufs[u % 2]
            rA, rB = (r0, r1) if u % 2 == 0 else (r1, r0)
            sA, sB = (sem0, sem1) if u % 2 == 0 else (sem1, sem0)
            if u == 0:
                pltpu.sync_copy(src_hbm.at[blk, 0], cs)
                pltpu.sync_copy(dst_hbm.at[blk, 0], cd)
                pltpu.async_copy(tab.at[cs.at[0]], rA, sA)
            if u + 1 < nsup:
                ns, nd = idx_bufs[(u + 1) % 2]
                pltpu.async_copy(src_hbm.at[blk, u + 1], ns, isem)
                pltpu.async_copy(dst_hbm.at[blk, u + 1], nd, isem)

            def body(m, carry, cs=cs, cd=cd, rA=rA, rB=rB, sA=sA, sB=sB):
                j0 = 2 * m
                pltpu.make_async_copy(tab.at[cs.at[j0]], rA, sA).wait()
                pltpu.async_copy(tab.at[cs.at[j0 + 1]], rB, sB)
                pltpu.sync_copy(rA, acc.at[cd.at[j0]], add=True)
                pltpu.make_async_copy(tab.at[cs.at[j0 + 1]], rB, sB).wait()
                pltpu.async_copy(tab.at[cs.at[j0 + 2]], rA, sA)
                pltpu.sync_copy(rB, acc.at[cd.at[j0 + 1]], add=True)
                return carry

            lax.fori_loop(0, (sch - 1) // 2, body, 0)
            pltpu.make_async_copy(tab.at[cs.at[sch - 1]], rA, sA).wait()
            if u + 1 < nsup:
                ns, nd = idx_bufs[(u + 1) % 2]
                pltpu.make_async_copy(src_hbm.at[blk, u + 1], ns, isem).wait()
                pltpu.make_async_copy(dst_hbm.at[blk, u + 1], nd, isem).wait()
                pltpu.async_copy(tab.at[ns.at[0]], rB, sB)
            pltpu.sync_copy(rA, acc.at[cd.at[sch - 1]], add=True)

        plsc.subcore_barrier()

        @pl.when(c == 0)
        def _():
            pltpu.sync_copy(acc.at[pl.ds(s * 640, 640)], out0.at[pl.ds(s * 640, 640)])

        @pl.when(c == 1)
        def _():
            pltpu.sync_copy(acc.at[pl.ds(s * 640, 640)], out1.at[pl.ds(s * 640, 640)])

    return k


_sc_scatter128 = _make_sc_scatter(128, 25)
_sc_scatter64 = _make_sc_scatter(64, 125, spmem_table=True)


def _dinv_col(deg_ref):
    deg = deg_ref[0] + deg_ref[1] + 1.0
    return lax.rsqrt(deg)


def _y1_body(x_ref, w_ref, deg_ref, y_ref):
    dinv = _dinv_col(deg_ref)
    xw = jnp.dot(x_ref[...], w_ref[...], preferred_element_type=jnp.float32)
    y_ref[...] = xw * dinv


def _y2_body(a0_ref, a1_ref, y1_ref, deg_ref, b1_ref, w2_ref, y2_ref):
    dinv = _dinv_col(deg_ref)
    h = dinv * (a0_ref[...] + a1_ref[...] + y1_ref[...]) + b1_ref[...]
    h = jnp.maximum(h, 0.0)
    y2_ref[...] = jnp.dot(h, w2_ref[...], preferred_element_type=jnp.float32) * dinv


def _final_body(a0_ref, a1_ref, y2_ref, deg_ref, b2_ref, batch_ref, lat_ref,
                sum_ref, cnt_ref):
    i = pl.program_id(0)
    dinv = _dinv_col(deg_ref)
    out2 = dinv * (a0_ref[...] + a1_ref[...] + y2_ref[...]) + b2_ref[...]
    bb = batch_ref[...]
    gid = lax.broadcasted_iota(jnp.int32, (1, NG), 1).astype(jnp.float32)
    oht = (bb == gid).astype(jnp.float32)
    dn = (((0,), (0,)), ((), ()))
    contrib = lax.dot_general(oht, out2, dn,
                              preferred_element_type=jnp.float32)
    cnt = lax.dot_general(oht, jnp.ones((BK, 1), jnp.float32), dn,
                          preferred_element_type=jnp.float32)

    @pl.when(i == 0)
    def _():
        sum_ref[...] = jnp.zeros_like(sum_ref)
        cnt_ref[...] = jnp.zeros_like(cnt_ref)

    sum_ref[...] += contrib
    cnt_ref[...] += jnp.broadcast_to(cnt, (NG, 64))

    @pl.when(i == G - 1)
    def _():
        lat_ref[...] = sum_ref[...] / jnp.maximum(cnt_ref[...], 1.0)


_row_spec = lambda d: pl.BlockSpec((BK, d), lambda i: (i, 0))
_deg_spec = pl.BlockSpec((2, BK, 1), lambda i: (0, i, 0))


def _full(shape):
    return pl.BlockSpec(shape, lambda *_: tuple(0 for _ in shape))


_y1_call = pl.pallas_call(
    _y1_body, grid=(G,),
    in_specs=[_row_spec(128), _full((128, 128)), _deg_spec],
    out_specs=_row_spec(128),
    out_shape=jax.ShapeDtypeStruct((NP, 128), jnp.float32),
)

_y2_call = pl.pallas_call(
    _y2_body, grid=(G,),
    in_specs=[_row_spec(128), _row_spec(128), _row_spec(128), _deg_spec,
              _full((1, 128)), _full((128, 64))],
    out_specs=_row_spec(64),
    out_shape=jax.ShapeDtypeStruct((NP, 64), jnp.float32),
)

_final_call = pl.pallas_call(
    _final_body, grid=(G,),
    in_specs=[_row_spec(64), _row_spec(64), _row_spec(64), _deg_spec,
              _full((1, 64)), pl.BlockSpec((BK, 1), lambda i: (i, 0))],
    out_specs=_full((NG, 64)),
    out_shape=jax.ShapeDtypeStruct((NG, 64), jnp.float32),
    scratch_shapes=[pltpu.VMEM((NG, 64), jnp.float32),
                    pltpu.VMEM((NG, 64), jnp.float32)],
)


def kernel(x, edge_index, batch, W1, b1, W2, b2):
    x = x.astype(jnp.float32)
    ei = edge_index.astype(jnp.int32)
    src_a = ei[0].reshape(NW, NCHUNK // 25, 25, C)
    dst_a = ei[1].reshape(NW, NCHUNK // 25, 25, C)
    src_b = ei[0].reshape(NW, 1, NCHUNK, C)
    dst_b = ei[1].reshape(NW, 1, NCHUNK, C)
    dst_flat = ei[1].reshape(NW, NCHUNK, C)
    x_p = jnp.pad(x, ((0, NP - N), (0, 0)))
    batchf = jnp.pad(batch.astype(jnp.float32), (0, NP - N),
                     constant_values=float(NG)).reshape(NP, 1)

    degp = _sc_degree(dst_flat).reshape(2, NP, 1)
    y1 = _y1_call(x_p, W1, degp)
    a10, a11 = _sc_scatter128(y1, src_a, dst_a)
    y2 = _y2_call(a10, a11, y1, degp, b1.reshape(1, 128), W2)
    a20, a21 = _sc_scatter64(y2, src_b, dst_b)
    return _final_call(a20, a21, y2, degp, b2.reshape(1, 64), batchf)
